# baseline (speedup 1.0000x reference)
import jax
import jax.numpy as jnp
from jax.experimental import pallas as pl
from jax.experimental.pallas import tpu as pltpu

N_EDGES = 20000
D = 256
BLK = 5000
NB = N_EDGES // BLK


def _mp_kernel(x_ref, w3t_ref, w3b_ref, b3_ref, out_ref,
               xs_scr, pooled_scr, cvec_scr):
    p = pl.program_id(0)
    j = pl.program_id(1)

    @pl.when(p == 0)
    def _phase_stream():
        xs_scr[pl.ds(j * BLK, BLK), :] = x_ref[...].astype(jnp.bfloat16)
        blk_max = jnp.max(x_ref[...], axis=0, keepdims=True)

        @pl.when(j == 0)
        def _():
            pooled_scr[...] = blk_max

        @pl.when(j > 0)
        def _():
            pooled_scr[...] = jnp.maximum(pooled_scr[...], blk_max)

    @pl.when(p == 1)
    def _phase_emit():
        @pl.when(j == 0)
        def _():
            cvec_scr[...] = (
                jnp.dot(pooled_scr[...], w3b_ref[...],
                        preferred_element_type=jnp.float32)
                + b3_ref[...]
            )

        y = jnp.dot(xs_scr[pl.ds(j * BLK, BLK), :], w3t_ref[...],
                    preferred_element_type=jnp.float32)
        out_ref[...] = jnp.maximum(y + cvec_scr[...], 0.0)


def kernel(edge_pred, edge_corner, all_corners, edge_x, image_x, W3, b3,
           interpret=False):
    del edge_pred, edge_corner, all_corners, image_x
    w3t = W3[:D, :].astype(jnp.bfloat16)
    w3b = W3[D:, :]
    b3_2d = b3.reshape(1, D)

    out = pl.pallas_call(
        _mp_kernel,
        grid=(2, NB),
        in_specs=[
            pl.BlockSpec((BLK, D), lambda p, j: (jnp.where(p == 0, j, NB - 1), 0)),
            pl.BlockSpec((D, D), lambda p, j: (0, 0)),
            pl.BlockSpec((D, D), lambda p, j: (0, 0)),
            pl.BlockSpec((1, D), lambda p, j: (0, 0)),
        ],
        out_specs=pl.BlockSpec((BLK, D), lambda p, j: (p * j, 0)),
        out_shape=jax.ShapeDtypeStruct((N_EDGES, D), jnp.float32),
        scratch_shapes=[
            pltpu.VMEM((N_EDGES, D), jnp.bfloat16),
            pltpu.VMEM((1, D), jnp.float32),
            pltpu.VMEM((1, D), jnp.float32),
        ],
        interpret=interpret,
    )(edge_x, w3t, w3b, b3_2d)
    return out


# speedup vs baseline: 1.1271x; 1.1271x over previous
import jax
import jax.numpy as jnp
from jax.experimental import pallas as pl
from jax.experimental.pallas import tpu as pltpu

N_EDGES = 20000
D = 256
BLK = 5000
NB = N_EDGES // BLK


def _mp_kernel(x_ref, w3t_ref, w3b_ref, b3_ref, out_ref,
               z_scr, pooled_scr, cvec_scr):
    p = pl.program_id(0)
    j = pl.program_id(1)

    @pl.when(p == 0)
    def _phase_stream():
        z_scr[pl.ds(j * BLK, BLK), :] = jnp.dot(
            x_ref[...], w3t_ref[...], preferred_element_type=jnp.float32)
        blk_max = jnp.max(x_ref[...], axis=0, keepdims=True)

        @pl.when(j == 0)
        def _():
            pooled_scr[...] = blk_max

        @pl.when(j > 0)
        def _():
            pooled_scr[...] = jnp.maximum(pooled_scr[...], blk_max)

        @pl.when(j == NB - 1)
        def _():
            cvec_scr[...] = (
                jnp.dot(pooled_scr[...], w3b_ref[...],
                        preferred_element_type=jnp.float32)
                + b3_ref[...]
            )

    @pl.when(p == 1)
    def _phase_emit():
        out_ref[...] = jnp.maximum(
            z_scr[pl.ds(j * BLK, BLK), :] + cvec_scr[...], 0.0)


def kernel(edge_pred, edge_corner, all_corners, edge_x, image_x, W3, b3,
           interpret=False):
    del edge_pred, edge_corner, all_corners, image_x
    w3t = W3[:D, :]
    w3b = W3[D:, :]
    b3_2d = b3.reshape(1, D)

    out = pl.pallas_call(
        _mp_kernel,
        grid=(2, NB),
        in_specs=[
            pl.BlockSpec((BLK, D), lambda p, j: ((1 - p) * j + p * (NB - 1), 0)),
            pl.BlockSpec((D, D), lambda p, j: (0, 0)),
            pl.BlockSpec((D, D), lambda p, j: (0, 0)),
            pl.BlockSpec((1, D), lambda p, j: (0, 0)),
        ],
        out_specs=pl.BlockSpec((BLK, D), lambda p, j: (p * j, 0)),
        out_shape=jax.ShapeDtypeStruct((N_EDGES, D), jnp.float32),
        scratch_shapes=[
            pltpu.VMEM((N_EDGES, D), jnp.float32),
            pltpu.VMEM((1, D), jnp.float32),
            pltpu.VMEM((1, D), jnp.float32),
        ],
        interpret=interpret,
    )(edge_x, w3t, w3b, b3_2d)
    return out
